# Initial kernel scaffold; baseline (speedup 1.0000x reference)
import jax
import jax.numpy as jnp
from jax.experimental import pallas as pl


def kernel(x, edge_index, batch, W, a_src, a_dst, bias, bn_gamma, bn_beta, fc_W, fc_b):
    raise NotImplementedError("write your pallas kernel here")


# speedup vs baseline: 73.7510x; 73.7510x over previous
import functools

import jax
import jax.numpy as jnp
from jax import lax
from jax.experimental import pallas as pl
from jax.experimental.pallas import tpu as pltpu
from jax.experimental.pallas import tpu_sc as plsc

_N = 10000
_E = 320000
_G = 64
_DIN = 128
_H = 2
_C = 64
_HC = _H * _C
_LATENT = 64

_NC = 2
_NS = 16
_NW = _NC * _NS
_EC = _E // _NW
_CH = 80
_NCH = _EC // _CH
_RW = 144
_NPAD = 10240
_RPT = _NPAD // _NS

_RPRE = 1000
_RPOST = 1024


def _pre_body(x_ref, w_ref, a4_ref, h_ref, al_ref, m_ref):
    i = pl.program_id(0)
    h = jnp.dot(x_ref[...], w_ref[...], preferred_element_type=jnp.float32)
    h_ref[...] = jnp.concatenate(
        [h, jnp.zeros((_RPRE, _RW - _HC), jnp.float32)], axis=1)
    al = jnp.dot(h, a4_ref[...], preferred_element_type=jnp.float32)
    al_ref[...] = al
    bm = jnp.broadcast_to(jnp.max(al, axis=0)[None, :], (8, 16))

    @pl.when(i == 0)
    def _():
        m_ref[...] = bm

    @pl.when(i > 0)
    def _():
        m_ref[...] = jnp.maximum(m_ref[...], bm)


def _sc_body(src_hbm, dst_hbm, atab_hbm, m_hbm, hfeat_hbm, out_hbm,
             acc_sh, sidx_v, didx_v, sbuf_v, dbuf_v, m_v, gbuf_v,
             gsem0, gsem1):
    cid = lax.axis_index("c")
    sid = lax.axis_index("s")
    wid = sid * _NC + cid

    pltpu.sync_copy(m_hbm, m_v)

    zvec = jnp.zeros((16,), jnp.float32)

    def _zero_row(r, _):
        for c9 in range(_RW // 16):
            gbuf_v[0, r, pl.ds(c9 * 16, 16)] = zvec
        return 0

    lax.fori_loop(0, _CH, _zero_row, 0)
    for z in range(_RPT // _CH):
        pltpu.sync_copy(gbuf_v.at[0],
                        acc_sh.at[pl.ds(sid * _RPT + z * _CH, _CH)])
    plsc.subcore_barrier()

    mv = m_v[...]
    m0 = mv[0]
    m1 = mv[1]
    c0 = jnp.full((16,), 0, jnp.int32)
    c1 = jnp.full((16,), 1, jnp.int32)
    c2 = jnp.full((16,), 2, jnp.int32)
    c3 = jnp.full((16,), 3, jnp.int32)
    c128 = jnp.full((16,), 128, jnp.int32)
    c129 = jnp.full((16,), 129, jnp.int32)
    iota16 = lax.iota(jnp.int32, 16)
    cb = [jnp.full((16,), 0, jnp.int32), jnp.full((16,), 1, jnp.int32)]

    def _load_idx(j, buf):
        pltpu.sync_copy(src_hbm.at[wid, j], sidx_v.at[buf])
        pltpu.sync_copy(dst_hbm.at[wid, j], didx_v.at[buf])

    def _start_gather(buf):
        pltpu.async_copy(hfeat_hbm.at[sidx_v.at[buf]], gbuf_v.at[buf],
                         gsem0 if buf == 0 else gsem1)

    def _process(j, buf, jpre, prefetch):
        pltpu.make_async_copy(hfeat_hbm.at[sidx_v.at[buf]], gbuf_v.at[buf],
                              gsem0 if buf == 0 else gsem1).wait()
        pltpu.sync_copy(atab_hbm.at[sidx_v.at[buf]], sbuf_v)
        pltpu.sync_copy(atab_hbm.at[didx_v.at[buf]], dbuf_v)

        def _group(i, _):
            rows16 = iota16 + i * 16
            e0 = (plsc.load_gather(sbuf_v, [rows16, c0])
                  + plsc.load_gather(dbuf_v, [rows16, c2]))
            e1 = (plsc.load_gather(sbuf_v, [rows16, c1])
                  + plsc.load_gather(dbuf_v, [rows16, c3]))
            e0 = jnp.where(e0 >= 0.0, e0, 0.2 * e0)
            e1 = jnp.where(e1 >= 0.0, e1, 0.2 * e1)
            w0 = jnp.exp(e0 - m0)
            w1 = jnp.exp(e1 - m1)
            plsc.store_scatter(gbuf_v, [cb[buf], rows16, c128], w0)
            plsc.store_scatter(gbuf_v, [cb[buf], rows16, c129], w1)
            for e16 in range(16):
                w0s = w0[e16]
                w1s = w1[e16]
                r = i * 16 + e16
                for c in range(4):
                    gbuf_v[buf, r, pl.ds(c * 16, 16)] = (
                        gbuf_v[buf, r, pl.ds(c * 16, 16)] * w0s)
                for c in range(4, 8):
                    gbuf_v[buf, r, pl.ds(c * 16, 16)] = (
                        gbuf_v[buf, r, pl.ds(c * 16, 16)] * w1s)
            return 0

        lax.fori_loop(0, _CH // 16, _group, 0)
        pltpu.sync_copy(gbuf_v.at[buf], acc_sh.at[didx_v.at[buf]], add=True)
        if prefetch:
            _load_idx(jpre, buf)
            _start_gather(buf)

    _load_idx(0, 0)
    _load_idx(1, 1)
    _start_gather(0)
    _start_gather(1)

    def _pair(j2, _):
        j = 2 * j2
        _process(j, 0, j + 2, True)
        _process(j + 1, 1, j + 3, True)
        return 0

    lax.fori_loop(0, (_NCH - 3) // 2, _pair, 0)
    _process(_NCH - 3, 0, _NCH - 1, True)
    _process(_NCH - 2, 1, 0, False)
    _process(_NCH - 1, 0, 0, False)

    plsc.subcore_barrier()
    base = cid * _NPAD + sid * _RPT
    pltpu.sync_copy(acc_sh.at[pl.ds(sid * _RPT, _RPT)],
                    out_hbm.at[pl.ds(base, _RPT)])


def _post_body(a0_ref, a1_ref, b_ref, bias_ref, g_ref, be_ref, fw_ref,
               fb_ref, out_ref, pooled_s):
    i = pl.program_id(0)
    a = a0_ref[...]
    b = a1_ref[...]
    feats = a[:, 0:128] + b[:, 0:128]
    d0 = a[:, 128:129] + b[:, 128:129]
    d1 = a[:, 129:130] + b[:, 129:130]
    den = jnp.concatenate([jnp.broadcast_to(d0, (_RPOST, 64)),
                           jnp.broadcast_to(d1, (_RPOST, 64))], axis=1)
    node = feats / (den + 1e-16) + bias_ref[...]
    bb = b_ref[0, 0, :]
    mask = (lax.broadcasted_iota(jnp.int32, (_G, _RPOST), 0)
            == bb[None, :]).astype(jnp.float32)
    contrib = jnp.dot(mask, node, preferred_element_type=jnp.float32)

    @pl.when(i == 0)
    def _():
        pooled_s[...] = contrib

    @pl.when(i > 0)
    def _():
        pooled_s[...] = pooled_s[...] + contrib

    @pl.when(i == (_NPAD // _RPOST) - 1)
    def _():
        pooled = pooled_s[...]
        mean = jnp.mean(pooled, axis=0, keepdims=True)
        var = jnp.mean((pooled - mean) ** 2, axis=0, keepdims=True)
        normed = ((pooled - mean) / jnp.sqrt(var + 1e-5) * g_ref[...]
                  + be_ref[...])
        out_ref[...] = (jnp.dot(normed, fw_ref[...],
                                preferred_element_type=jnp.float32)
                        + fb_ref[...])


def kernel(x, edge_index, batch, W, a_src, a_dst, bias, bn_gamma, bn_beta,
           fc_W, fc_b):
    f32 = jnp.float32
    z64 = jnp.zeros((_C,), f32)
    a4 = jnp.stack([
        jnp.concatenate([a_src[0], z64]),
        jnp.concatenate([z64, a_src[1]]),
        jnp.concatenate([a_dst[0], z64]),
        jnp.concatenate([z64, a_dst[1]]),
    ], axis=1)
    a16 = jnp.concatenate([a4, jnp.zeros((_DIN, 12), f32)], axis=1)

    npre = _N // _RPRE
    h, atab, mmax = pl.pallas_call(
        _pre_body,
        grid=(npre,),
        in_specs=[
            pl.BlockSpec((_RPRE, _DIN), lambda i: (i, 0)),
            pl.BlockSpec((_DIN, _DIN), lambda i: (0, 0)),
            pl.BlockSpec((_DIN, 16), lambda i: (0, 0)),
        ],
        out_specs=[
            pl.BlockSpec((_RPRE, _RW), lambda i: (i, 0)),
            pl.BlockSpec((_RPRE, 16), lambda i: (i, 0)),
            pl.BlockSpec((8, 16), lambda i: (0, 0)),
        ],
        out_shape=[
            jax.ShapeDtypeStruct((_N, _RW), f32),
            jax.ShapeDtypeStruct((_N, 16), f32),
            jax.ShapeDtypeStruct((8, 16), f32),
        ],
    )(x, W, a16)

    msum = mmax[0, 0:2] + mmax[0, 2:4]
    mbound = jnp.where(msum >= 0.0, msum, 0.2 * msum)
    mvec = jnp.zeros((16,), f32).at[0:2].set(mbound)

    src3 = edge_index[0].reshape(_NW, _NCH, _CH)
    dst3 = edge_index[1].reshape(_NW, _NCH, _CH)

    mesh = plsc.VectorSubcoreMesh(core_axis_name="c", subcore_axis_name="s")
    acc2 = pl.kernel(
        _sc_body,
        out_type=jax.ShapeDtypeStruct((_NC * _NPAD, _RW), f32),
        mesh=mesh,
        compiler_params=pltpu.CompilerParams(use_tc_tiling_on_sc=False,
                                             needs_layout_passes=False),
        scratch_types=[
            pltpu.VMEM_SHARED((_NPAD, _RW), f32),
            pltpu.VMEM((2, _CH), jnp.int32),
            pltpu.VMEM((2, _CH), jnp.int32),
            pltpu.VMEM((_CH, 16), f32),
            pltpu.VMEM((_CH, 16), f32),
            pltpu.VMEM((16,), f32),
            pltpu.VMEM((2, _CH, _RW), f32),
            pltpu.SemaphoreType.DMA,
            pltpu.SemaphoreType.DMA,
        ],
    )(src3, dst3, atab, mvec, h)

    npost = _NPAD // _RPOST
    batch3 = jnp.concatenate(
        [batch, jnp.full((_NPAD - _N,), _G, jnp.int32)]
    ).reshape(npost, 1, _RPOST)
    out = pl.pallas_call(
        _post_body,
        grid=(npost,),
        in_specs=[
            pl.BlockSpec((_RPOST, _RW), lambda i: (i, 0)),
            pl.BlockSpec((_RPOST, _RW), lambda i: (i + npost, 0)),
            pl.BlockSpec((1, 1, _RPOST), lambda i: (i, 0, 0)),
            pl.BlockSpec((1, _HC), lambda i: (0, 0)),
            pl.BlockSpec((1, _HC), lambda i: (0, 0)),
            pl.BlockSpec((1, _HC), lambda i: (0, 0)),
            pl.BlockSpec((_HC, _LATENT), lambda i: (0, 0)),
            pl.BlockSpec((1, _LATENT), lambda i: (0, 0)),
        ],
        out_specs=pl.BlockSpec((_G, _LATENT), lambda i: (0, 0)),
        out_shape=jax.ShapeDtypeStruct((_G, _LATENT), f32),
        scratch_shapes=[pltpu.VMEM((_G, _HC), f32)],
    )(acc2, acc2, batch3, bias.reshape(1, _HC), bn_gamma.reshape(1, _HC),
      bn_beta.reshape(1, _HC), fc_W, fc_b.reshape(1, _LATENT))
    return out
